# baseline (speedup 1.0000x reference)
import functools

import jax
import jax.numpy as jnp
from jax import lax
from jax.experimental import pallas as pl
from jax.experimental.pallas import tpu as pltpu
from jax.experimental.pallas import tpu_sc as plsc

N_USER = 50000
N_MOVIE = 10000
E = 800000
H = 64
Q = 16

CHUNK = 80
NBLK = 5
BLKROWS = 125

U_ST, U_LA = 3120, 3200
M_ST, M_LA = 640, 400
U_ROWS = U_LA // CHUNK
M_ROWS = M_LA * 0 + 8

_f32 = jnp.float32


def _fill_rows(ref, nrows, width, vec16):
    for r in range(nrows):
        for h in range(width // 16):
            ref[r, pl.ds(h * 16, 16)] = vec16


def _per_tile(s, st, la, fn):
    @pl.when(s < 15)
    def _():
        fn(s * st, st // CHUNK)

    @pl.when(s == 15)
    def _():
        fn(15 * st, la // CHUNK)


def _ind_zero(zbuf, sh, iot, nch):
    for k in range(nch):
        pltpu.sync_copy(zbuf, sh.at[iot.at[k]])


def _ind_stage(vbuf, hbm, sh, iot, base, nch):
    for k in range(nch):
        pltpu.sync_copy(hbm.at[pl.ds(base + k * CHUNK, CHUNK)], vbuf)
        pltpu.sync_copy(vbuf, sh.at[iot.at[k]])


def _ind_drain(vbuf, sh, out3, iot, s, nch, sem):
    for k in range(nch):
        pltpu.async_copy(sh.at[iot.at[k]], vbuf, sem).wait()
        pltpu.sync_copy(vbuf, out3.at[s, pl.ds(k * CHUNK, CHUNK)])


def _counts_body(src4, dst4, iou, iom, cu3, cm3,
                 cu_sh, cm_sh, idxb, iotu, iotm, ones, vbuf, sem):
    c = lax.axis_index("c")
    s = lax.axis_index("s")
    one16 = jnp.ones((16,), _f32)
    zero16 = jnp.zeros((16,), _f32)
    _fill_rows(ones, CHUNK, 16, one16)
    _fill_rows(vbuf, CHUNK, 16, zero16)
    pltpu.sync_copy(iou.at[s], iotu)
    pltpu.sync_copy(iom.at[s], iotm)

    @pl.when(c == 0)
    def _():
        _per_tile(s, U_ST, U_LA, lambda b, n: _ind_zero(vbuf, cu_sh, iotu, n))

    @pl.when(c == 1)
    def _():
        _per_tile(s, M_ST, M_LA, lambda b, n: _ind_zero(vbuf, cm_sh, iotm, n))
    plsc.subcore_barrier()

    def scatter(idx_hbm, cnt_sh):
        def blk_body(blk, carry):
            pltpu.sync_copy(idx_hbm.at[s, blk], idxb)
            for j in range(BLKROWS):
                pltpu.sync_copy(ones, cnt_sh.at[idxb.at[j]], add=True)
            return carry
        lax.fori_loop(0, NBLK, blk_body, 0)

    @pl.when(c == 0)
    def _():
        scatter(src4, cu_sh)

    @pl.when(c == 1)
    def _():
        scatter(dst4, cm_sh)
    plsc.subcore_barrier()

    @pl.when(c == 0)
    def _():
        _per_tile(s, U_ST, U_LA,
                  lambda b, n: _ind_drain(vbuf, cu_sh, cu3, iotu, s, n, sem))

    @pl.when(c == 1)
    def _():
        _per_tile(s, M_ST, M_LA,
                  lambda b, n: _ind_drain(vbuf, cm_sh, cm3, iotm, s, n, sem))


@functools.partial(
    pl.kernel,
    out_type=[jax.ShapeDtypeStruct((16, U_LA, 16), _f32),
              jax.ShapeDtypeStruct((16, M_ROWS * CHUNK, 16), _f32)],
    mesh=plsc.VectorSubcoreMesh(core_axis_name="c", subcore_axis_name="s"),
    scratch_types=[
        pltpu.VMEM_SHARED((N_USER, 16), _f32),
        pltpu.VMEM_SHARED((N_MOVIE, 16), _f32),
        pltpu.VMEM((BLKROWS, CHUNK), jnp.int32),
        pltpu.VMEM((U_ROWS, CHUNK), jnp.int32),
        pltpu.VMEM((M_ROWS, CHUNK), jnp.int32),
        pltpu.VMEM((CHUNK, 16), _f32),
        pltpu.VMEM((CHUNK, 16), _f32),
        pltpu.SemaphoreType.DMA,
    ],
)
def _sc_counts(src4, dst4, iou, iom, cu3, cm3,
               cu_sh, cm_sh, idxb, iotu, iotm, ones, vbuf, sem):
    _counts_body(src4, dst4, iou, iom, cu3, cm3,
                 cu_sh, cm_sh, idxb, iotu, iotm, ones, vbuf, sem)


def _dir_body(gidx_hbm, sidx_hbm, io_t, io_a, tq_a, tq_b, acc_a, acc_b,
              tbl_sh, acc_sh, gidx, sidx, iott, iota, buf, buf2, vbuf,
              sem0, sem1, sem2,
              t_st, t_la, a_st, a_la):
    c = lax.axis_index("c")
    s = lax.axis_index("s")
    zero16 = jnp.zeros((16,), _f32)
    _fill_rows(vbuf, CHUNK, Q, zero16)
    pltpu.sync_copy(io_t.at[s], iott)
    pltpu.sync_copy(io_a.at[s], iota)

    _per_tile(s, a_st, a_la, lambda b, n: _ind_zero(vbuf, acc_sh, iota, n))

    @pl.when(c == 0)
    def _():
        _per_tile(s, t_st, t_la,
                  lambda b, n: _ind_stage(buf, tq_a, tbl_sh, iott, b, n))

    @pl.when(c == 1)
    def _():
        _per_tile(s, t_st, t_la,
                  lambda b, n: _ind_stage(buf, tq_b, tbl_sh, iott, b, n))
    plsc.subcore_barrier()

    bufs = (buf, buf2)
    sems = (sem0, sem2)

    def blk_body(blk, carry):
        pltpu.sync_copy(gidx_hbm.at[s, blk], gidx)
        pltpu.sync_copy(sidx_hbm.at[s, blk], sidx)
        pltpu.async_copy(tbl_sh.at[gidx.at[0]], bufs[0], sems[0])
        for j in range(BLKROWS):
            b = j % 2
            pltpu.make_async_copy(tbl_sh.at[gidx.at[j]], bufs[b],
                                  sems[b]).wait()
            if j >= 1:
                pltpu.make_async_copy(bufs[1 - b],
                                      acc_sh.at[sidx.at[j - 1]], sem1).wait()
            if j + 1 < BLKROWS:
                pltpu.async_copy(tbl_sh.at[gidx.at[j + 1]], bufs[1 - b],
                                 sems[1 - b])
            pltpu.async_copy(bufs[b], acc_sh.at[sidx.at[j]], sem1, add=True)
        bl = (BLKROWS - 1) % 2
        pltpu.make_async_copy(bufs[bl],
                              acc_sh.at[sidx.at[BLKROWS - 1]], sem1).wait()
        return carry
    lax.fori_loop(0, NBLK, blk_body, 0)
    plsc.subcore_barrier()

    @pl.when(c == 0)
    def _():
        _per_tile(s, a_st, a_la,
                  lambda b, n: _ind_drain(vbuf, acc_sh, acc_a, iota, s, n, sem1))

    @pl.when(c == 1)
    def _():
        _per_tile(s, a_st, a_la,
                  lambda b, n: _ind_drain(vbuf, acc_sh, acc_b, iota, s, n, sem1))


def _make_dir_kernel(tbl_rows, acc_rows, t_stripes, a_stripes, a_rows, t_rows):
    @functools.partial(
        pl.kernel,
        out_type=[jax.ShapeDtypeStruct((16, a_rows * CHUNK, Q), _f32),
                  jax.ShapeDtypeStruct((16, a_rows * CHUNK, Q), _f32)],
        mesh=plsc.VectorSubcoreMesh(core_axis_name="c", subcore_axis_name="s"),
        scratch_types=[
            pltpu.VMEM_SHARED((tbl_rows, Q), _f32),
            pltpu.VMEM_SHARED((acc_rows, Q), _f32),
            pltpu.VMEM((BLKROWS, CHUNK), jnp.int32),
            pltpu.VMEM((BLKROWS, CHUNK), jnp.int32),
            pltpu.VMEM((t_rows, CHUNK), jnp.int32),
            pltpu.VMEM((a_rows, CHUNK), jnp.int32),
            pltpu.VMEM((CHUNK, Q), _f32),
            pltpu.VMEM((CHUNK, Q), _f32),
            pltpu.VMEM((CHUNK, Q), _f32),
            pltpu.SemaphoreType.DMA,
            pltpu.SemaphoreType.DMA,
            pltpu.SemaphoreType.DMA,
        ],
    )
    def _k(gidx_hbm, sidx_hbm, io_t, io_a, tq_a, tq_b, acc_a, acc_b,
           tbl_sh, acc_sh, gidx, sidx, iott, iota, buf, buf2, vbuf,
           sem0, sem1, sem2):
        _dir_body(gidx_hbm, sidx_hbm, io_t, io_a, tq_a, tq_b, acc_a, acc_b,
                  tbl_sh, acc_sh, gidx, sidx, iott, iota, buf, buf2, vbuf,
                  sem0, sem1, sem2,
                  t_stripes[0], t_stripes[1], a_stripes[0], a_stripes[1])
    return _k


_sc_agg_m = _make_dir_kernel(N_USER, N_MOVIE, (U_ST, U_LA), (M_ST, M_LA),
                             M_ROWS, U_ROWS)
_sc_agg_u = _make_dir_kernel(N_MOVIE, N_USER, (M_ST, M_LA), (U_ST, U_LA),
                             U_ROWS, M_ROWS)


_BN = 1000


def _ln(o, g, b):
    m = jnp.mean(o, axis=-1, keepdims=True)
    v = jnp.mean((o - m) ** 2, axis=-1, keepdims=True)
    return (o - m) / jnp.sqrt(v + 1e-5) * g + b


def _q_split_store(on, outs):
    for k in range(4):
        outs[k][...] = on[:, k * Q:(k + 1) * Q]


def _enc(x, W, b, g, be):
    N, F = x.shape

    def body(x_ref, w_ref, b_ref, g_ref, be_ref, *outs):
        h = jnp.dot(x_ref[...], w_ref[...], preferred_element_type=_f32)
        h = jax.nn.relu(h + b_ref[...])
        hn = _ln(h, g_ref[...], be_ref[...])
        _q_split_store(hn, outs)

    return pl.pallas_call(
        body,
        grid=(N // _BN,),
        in_specs=[
            pl.BlockSpec((_BN, F), lambda i: (i, 0)),
            pl.BlockSpec((F, H), lambda i: (0, 0)),
            pl.BlockSpec((1, H), lambda i: (0, 0)),
            pl.BlockSpec((1, H), lambda i: (0, 0)),
            pl.BlockSpec((1, H), lambda i: (0, 0)),
        ],
        out_specs=[pl.BlockSpec((_BN, Q), lambda i: (i, 0))] * 4,
        out_shape=[jax.ShapeDtypeStruct((N, Q), _f32)] * 4,
    )(x, W, b, g, be)


def _conv_dense(aq, cnt, hq, Wl, bl, Wr, g, b, relu, split):
    N = aq[0].shape[0]

    def body(a0r, a1r, a2r, a3r, cr, h0r, h1r, h2r, h3r,
             wlr, blr, wrr, gr, br, *outs):
        agg = jnp.concatenate([a0r[...], a1r[...], a2r[...], a3r[...]], axis=1)
        h = jnp.concatenate([h0r[...], h1r[...], h2r[...], h3r[...]], axis=1)
        c = cr[...][:, 0:1]
        mean = jnp.where(c > 0, agg / jnp.maximum(c, 1.0), 0.0)
        o = (jnp.dot(mean, wlr[...], preferred_element_type=_f32) + blr[...]
             + jnp.dot(h, wrr[...], preferred_element_type=_f32))
        on = _ln(o, gr[...], br[...])
        if relu:
            on = jax.nn.relu(on)
        if split:
            _q_split_store(on, outs)
        else:
            outs[0][...] = on

    if split:
        out_specs = [pl.BlockSpec((_BN, Q), lambda i: (i, 0))] * 4
        out_shape = [jax.ShapeDtypeStruct((N, Q), _f32)] * 4
    else:
        out_specs = [pl.BlockSpec((_BN, H), lambda i: (i, 0))]
        out_shape = [jax.ShapeDtypeStruct((N, H), _f32)]

    res = pl.pallas_call(
        body,
        grid=(N // _BN,),
        in_specs=(
            [pl.BlockSpec((_BN, Q), lambda i: (i, 0))] * 4
            + [pl.BlockSpec((_BN, 16), lambda i: (i, 0))]
            + [pl.BlockSpec((_BN, Q), lambda i: (i, 0))] * 4
            + [pl.BlockSpec((H, H), lambda i: (0, 0)),
               pl.BlockSpec((1, H), lambda i: (0, 0)),
               pl.BlockSpec((H, H), lambda i: (0, 0)),
               pl.BlockSpec((1, H), lambda i: (0, 0)),
               pl.BlockSpec((1, H), lambda i: (0, 0))]
        ),
        out_specs=out_specs,
        out_shape=out_shape,
    )(*aq, cnt, *hq, Wl, bl, Wr, g, b)
    return res if split else res[0]


def _assemble(o3, st, la):
    parts = [o3[t, :st] for t in range(15)] + [o3[15, :la]]
    return jnp.concatenate(parts, axis=0)


def _agg_all(src4, dst4, iou, iom, hq_u, hq_m):
    am = []
    for pair in ((0, 1), (2, 3)):
        a, b = _sc_agg_m(src4, dst4, iou, iom, hq_u[pair[0]], hq_u[pair[1]])
        am += [_assemble(a, M_ST, M_LA), _assemble(b, M_ST, M_LA)]
    au = []
    for pair in ((0, 1), (2, 3)):
        a, b = _sc_agg_u(dst4, src4, iom, iou, hq_m[pair[0]], hq_m[pair[1]])
        au += [_assemble(a, U_ST, U_LA), _assemble(b, U_ST, U_LA)]
    return tuple(au), tuple(am)


def kernel(x_user, x_movie, edge_src_user, edge_dst_movie, params):
    p = params
    r2 = lambda v: v.reshape(1, H)
    src4 = edge_src_user.astype(jnp.int32).reshape(16, NBLK, BLKROWS, CHUNK)
    dst4 = edge_dst_movie.astype(jnp.int32).reshape(16, NBLK, BLKROWS, CHUNK)
    iou = jnp.minimum(
        jnp.arange(16, dtype=jnp.int32)[:, None] * U_ST
        + jnp.arange(U_LA, dtype=jnp.int32)[None, :],
        N_USER - 1).reshape(16, U_ROWS, CHUNK)
    iom = jnp.minimum(
        jnp.arange(16, dtype=jnp.int32)[:, None] * M_ST
        + jnp.arange(M_ROWS * CHUNK, dtype=jnp.int32)[None, :],
        N_MOVIE - 1).reshape(16, M_ROWS, CHUNK)

    hq_u = _enc(x_user, p['W_ue'], r2(p['b_ue']), r2(p['g_ue']), r2(p['be_ue']))
    hq_m = _enc(x_movie, p['W_me'], r2(p['b_me']), r2(p['g_me']), r2(p['be_me']))
    cu3, cm3 = _sc_counts(src4, dst4, iou, iom)
    cu = _assemble(cu3, U_ST, U_LA)
    cm = _assemble(cm3, M_ST, M_LA)

    aq_u, aq_m = _agg_all(src4, dst4, iou, iom, hq_u, hq_m)
    h1q_u = _conv_dense(aq_u, cu, hq_u,
                        p['Wl1_u'], r2(p['bl1_u']), p['Wr1_u'],
                        r2(p['g1_u']), r2(p['b1_u']), relu=True, split=True)
    h1q_m = _conv_dense(aq_m, cm, hq_m,
                        p['Wl1_m'], r2(p['bl1_m']), p['Wr1_m'],
                        r2(p['g1_m']), r2(p['b1_m']), relu=True, split=True)

    bq_u, bq_m = _agg_all(src4, dst4, iou, iom, h1q_u, h1q_m)
    out_u = _conv_dense(bq_u, cu, h1q_u,
                        p['Wl2_u'], r2(p['bl2_u']), p['Wr2_u'],
                        r2(p['g2_u']), r2(p['b2_u']), relu=False, split=False)
    out_m = _conv_dense(bq_m, cm, h1q_m,
                        p['Wl2_m'], r2(p['bl2_m']), p['Wr2_m'],
                        r2(p['g2_m']), r2(p['b2_m']), relu=False, split=False)
    return out_u, out_m


# speedup vs baseline: 1.0055x; 1.0055x over previous
import functools

import jax
import jax.numpy as jnp
from jax import lax
from jax.experimental import pallas as pl
from jax.experimental.pallas import tpu as pltpu
from jax.experimental.pallas import tpu_sc as plsc

N_USER = 50000
N_MOVIE = 10000
E = 800000
H = 64
Q = 16

CHUNK = 80
NBLK = 5
BLKROWS = 125

U_ST, U_LA = 3120, 3200
M_ST, M_LA = 640, 400
U_ROWS = U_LA // CHUNK
M_ROWS = M_LA * 0 + 8

_f32 = jnp.float32


def _fill_rows(ref, nrows, width, vec16):
    for r in range(nrows):
        for h in range(width // 16):
            ref[r, pl.ds(h * 16, 16)] = vec16


def _per_tile(s, st, la, fn):
    @pl.when(s < 15)
    def _():
        fn(s * st, st // CHUNK)

    @pl.when(s == 15)
    def _():
        fn(15 * st, la // CHUNK)


def _ind_zero(zbuf, sh, iot, nch):
    for k in range(nch):
        pltpu.sync_copy(zbuf, sh.at[iot.at[k]])


def _ind_stage(vbuf, hbm, sh, iot, base, nch):
    for k in range(nch):
        pltpu.sync_copy(hbm.at[pl.ds(base + k * CHUNK, CHUNK)], vbuf)
        pltpu.sync_copy(vbuf, sh.at[iot.at[k]])


def _ind_drain(vbuf, sh, out3, iot, s, nch, sem):
    for k in range(nch):
        pltpu.async_copy(sh.at[iot.at[k]], vbuf, sem).wait()
        pltpu.sync_copy(vbuf, out3.at[s, pl.ds(k * CHUNK, CHUNK)])


def _counts_body(src4, dst4, iou, iom, cu3, cm3,
                 cu_sh, cm_sh, idxb, iotu, iotm, ones, vbuf, sem):
    c = lax.axis_index("c")
    s = lax.axis_index("s")
    one16 = jnp.ones((16,), _f32)
    zero16 = jnp.zeros((16,), _f32)
    _fill_rows(ones, CHUNK, 16, one16)
    _fill_rows(vbuf, CHUNK, 16, zero16)
    pltpu.sync_copy(iou.at[s], iotu)
    pltpu.sync_copy(iom.at[s], iotm)

    @pl.when(c == 0)
    def _():
        _per_tile(s, U_ST, U_LA, lambda b, n: _ind_zero(vbuf, cu_sh, iotu, n))

    @pl.when(c == 1)
    def _():
        _per_tile(s, M_ST, M_LA, lambda b, n: _ind_zero(vbuf, cm_sh, iotm, n))
    plsc.subcore_barrier()

    def scatter(idx_hbm, cnt_sh):
        def blk_body(blk, carry):
            pltpu.sync_copy(idx_hbm.at[s, blk], idxb)
            for j in range(BLKROWS):
                pltpu.sync_copy(ones, cnt_sh.at[idxb.at[j]], add=True)
            return carry
        lax.fori_loop(0, NBLK, blk_body, 0)

    @pl.when(c == 0)
    def _():
        scatter(src4, cu_sh)

    @pl.when(c == 1)
    def _():
        scatter(dst4, cm_sh)
    plsc.subcore_barrier()

    @pl.when(c == 0)
    def _():
        _per_tile(s, U_ST, U_LA,
                  lambda b, n: _ind_drain(vbuf, cu_sh, cu3, iotu, s, n, sem))

    @pl.when(c == 1)
    def _():
        _per_tile(s, M_ST, M_LA,
                  lambda b, n: _ind_drain(vbuf, cm_sh, cm3, iotm, s, n, sem))


@functools.partial(
    pl.kernel,
    out_type=[jax.ShapeDtypeStruct((16, U_LA, 16), _f32),
              jax.ShapeDtypeStruct((16, M_ROWS * CHUNK, 16), _f32)],
    mesh=plsc.VectorSubcoreMesh(core_axis_name="c", subcore_axis_name="s"),
    scratch_types=[
        pltpu.VMEM_SHARED((N_USER, 16), _f32),
        pltpu.VMEM_SHARED((N_MOVIE, 16), _f32),
        pltpu.VMEM((BLKROWS, CHUNK), jnp.int32),
        pltpu.VMEM((U_ROWS, CHUNK), jnp.int32),
        pltpu.VMEM((M_ROWS, CHUNK), jnp.int32),
        pltpu.VMEM((CHUNK, 16), _f32),
        pltpu.VMEM((CHUNK, 16), _f32),
        pltpu.SemaphoreType.DMA,
    ],
)
def _sc_counts(src4, dst4, iou, iom, cu3, cm3,
               cu_sh, cm_sh, idxb, iotu, iotm, ones, vbuf, sem):
    _counts_body(src4, dst4, iou, iom, cu3, cm3,
                 cu_sh, cm_sh, idxb, iotu, iotm, ones, vbuf, sem)


def _dir_body(gidx_hbm, sidx_hbm, io_t, io_a, tq_a, tq_b, acc_a, acc_b,
              tbl_sh, acc_sh, gidx, sidx, iott, iota, buf, buf2, vbuf,
              sem0, sem1, sem2,
              t_st, t_la, a_st, a_la):
    c = lax.axis_index("c")
    s = lax.axis_index("s")
    zero16 = jnp.zeros((16,), _f32)
    _fill_rows(vbuf, CHUNK, Q, zero16)
    pltpu.sync_copy(io_t.at[s], iott)
    pltpu.sync_copy(io_a.at[s], iota)

    _per_tile(s, a_st, a_la, lambda b, n: _ind_zero(vbuf, acc_sh, iota, n))

    @pl.when(c == 0)
    def _():
        _per_tile(s, t_st, t_la,
                  lambda b, n: _ind_stage(buf, tq_a, tbl_sh, iott, b, n))

    @pl.when(c == 1)
    def _():
        _per_tile(s, t_st, t_la,
                  lambda b, n: _ind_stage(buf, tq_b, tbl_sh, iott, b, n))
    plsc.subcore_barrier()

    bufs = (buf, buf2)
    sems = (sem0, sem2)

    def blk_body(blk, carry):
        pltpu.sync_copy(gidx_hbm.at[s, blk], gidx)
        pltpu.sync_copy(sidx_hbm.at[s, blk], sidx)
        pltpu.async_copy(tbl_sh.at[gidx.at[0]], bufs[0], sems[0])
        for j in range(BLKROWS):
            b = j % 2
            pltpu.make_async_copy(tbl_sh.at[gidx.at[j]], bufs[b],
                                  sems[b]).wait()
            if j + 1 < BLKROWS:
                pltpu.async_copy(tbl_sh.at[gidx.at[j + 1]], bufs[1 - b],
                                 sems[1 - b])
            pltpu.sync_copy(bufs[b], acc_sh.at[sidx.at[j]], add=True)
        return carry
    lax.fori_loop(0, NBLK, blk_body, 0)
    plsc.subcore_barrier()

    @pl.when(c == 0)
    def _():
        _per_tile(s, a_st, a_la,
                  lambda b, n: _ind_drain(vbuf, acc_sh, acc_a, iota, s, n, sem1))

    @pl.when(c == 1)
    def _():
        _per_tile(s, a_st, a_la,
                  lambda b, n: _ind_drain(vbuf, acc_sh, acc_b, iota, s, n, sem1))


def _make_dir_kernel(tbl_rows, acc_rows, t_stripes, a_stripes, a_rows, t_rows):
    @functools.partial(
        pl.kernel,
        out_type=[jax.ShapeDtypeStruct((16, a_rows * CHUNK, Q), _f32),
                  jax.ShapeDtypeStruct((16, a_rows * CHUNK, Q), _f32)],
        mesh=plsc.VectorSubcoreMesh(core_axis_name="c", subcore_axis_name="s"),
        scratch_types=[
            pltpu.VMEM_SHARED((tbl_rows, Q), _f32),
            pltpu.VMEM_SHARED((acc_rows, Q), _f32),
            pltpu.VMEM((BLKROWS, CHUNK), jnp.int32),
            pltpu.VMEM((BLKROWS, CHUNK), jnp.int32),
            pltpu.VMEM((t_rows, CHUNK), jnp.int32),
            pltpu.VMEM((a_rows, CHUNK), jnp.int32),
            pltpu.VMEM((CHUNK, Q), _f32),
            pltpu.VMEM((CHUNK, Q), _f32),
            pltpu.VMEM((CHUNK, Q), _f32),
            pltpu.SemaphoreType.DMA,
            pltpu.SemaphoreType.DMA,
            pltpu.SemaphoreType.DMA,
        ],
    )
    def _k(gidx_hbm, sidx_hbm, io_t, io_a, tq_a, tq_b, acc_a, acc_b,
           tbl_sh, acc_sh, gidx, sidx, iott, iota, buf, buf2, vbuf,
           sem0, sem1, sem2):
        _dir_body(gidx_hbm, sidx_hbm, io_t, io_a, tq_a, tq_b, acc_a, acc_b,
                  tbl_sh, acc_sh, gidx, sidx, iott, iota, buf, buf2, vbuf,
                  sem0, sem1, sem2,
                  t_stripes[0], t_stripes[1], a_stripes[0], a_stripes[1])
    return _k


_sc_agg_m = _make_dir_kernel(N_USER, N_MOVIE, (U_ST, U_LA), (M_ST, M_LA),
                             M_ROWS, U_ROWS)
_sc_agg_u = _make_dir_kernel(N_MOVIE, N_USER, (M_ST, M_LA), (U_ST, U_LA),
                             U_ROWS, M_ROWS)


_BN = 1000


def _ln(o, g, b):
    m = jnp.mean(o, axis=-1, keepdims=True)
    v = jnp.mean((o - m) ** 2, axis=-1, keepdims=True)
    return (o - m) / jnp.sqrt(v + 1e-5) * g + b


def _q_split_store(on, outs):
    for k in range(4):
        outs[k][...] = on[:, k * Q:(k + 1) * Q]


def _enc(x, W, b, g, be):
    N, F = x.shape

    def body(x_ref, w_ref, b_ref, g_ref, be_ref, *outs):
        h = jnp.dot(x_ref[...], w_ref[...], preferred_element_type=_f32)
        h = jax.nn.relu(h + b_ref[...])
        hn = _ln(h, g_ref[...], be_ref[...])
        _q_split_store(hn, outs)

    return pl.pallas_call(
        body,
        grid=(N // _BN,),
        in_specs=[
            pl.BlockSpec((_BN, F), lambda i: (i, 0)),
            pl.BlockSpec((F, H), lambda i: (0, 0)),
            pl.BlockSpec((1, H), lambda i: (0, 0)),
            pl.BlockSpec((1, H), lambda i: (0, 0)),
            pl.BlockSpec((1, H), lambda i: (0, 0)),
        ],
        out_specs=[pl.BlockSpec((_BN, Q), lambda i: (i, 0))] * 4,
        out_shape=[jax.ShapeDtypeStruct((N, Q), _f32)] * 4,
    )(x, W, b, g, be)


def _conv_dense(aq, cnt, hq, Wl, bl, Wr, g, b, relu, split):
    N = aq[0].shape[0]

    def body(a0r, a1r, a2r, a3r, cr, h0r, h1r, h2r, h3r,
             wlr, blr, wrr, gr, br, *outs):
        agg = jnp.concatenate([a0r[...], a1r[...], a2r[...], a3r[...]], axis=1)
        h = jnp.concatenate([h0r[...], h1r[...], h2r[...], h3r[...]], axis=1)
        c = cr[...][:, 0:1]
        mean = jnp.where(c > 0, agg / jnp.maximum(c, 1.0), 0.0)
        o = (jnp.dot(mean, wlr[...], preferred_element_type=_f32) + blr[...]
             + jnp.dot(h, wrr[...], preferred_element_type=_f32))
        on = _ln(o, gr[...], br[...])
        if relu:
            on = jax.nn.relu(on)
        if split:
            _q_split_store(on, outs)
        else:
            outs[0][...] = on

    if split:
        out_specs = [pl.BlockSpec((_BN, Q), lambda i: (i, 0))] * 4
        out_shape = [jax.ShapeDtypeStruct((N, Q), _f32)] * 4
    else:
        out_specs = [pl.BlockSpec((_BN, H), lambda i: (i, 0))]
        out_shape = [jax.ShapeDtypeStruct((N, H), _f32)]

    res = pl.pallas_call(
        body,
        grid=(N // _BN,),
        in_specs=(
            [pl.BlockSpec((_BN, Q), lambda i: (i, 0))] * 4
            + [pl.BlockSpec((_BN, 16), lambda i: (i, 0))]
            + [pl.BlockSpec((_BN, Q), lambda i: (i, 0))] * 4
            + [pl.BlockSpec((H, H), lambda i: (0, 0)),
               pl.BlockSpec((1, H), lambda i: (0, 0)),
               pl.BlockSpec((H, H), lambda i: (0, 0)),
               pl.BlockSpec((1, H), lambda i: (0, 0)),
               pl.BlockSpec((1, H), lambda i: (0, 0))]
        ),
        out_specs=out_specs,
        out_shape=out_shape,
    )(*aq, cnt, *hq, Wl, bl, Wr, g, b)
    return res if split else res[0]


def _assemble(o3, st, la):
    parts = [o3[t, :st] for t in range(15)] + [o3[15, :la]]
    return jnp.concatenate(parts, axis=0)


def _agg_all(src4, dst4, iou, iom, hq_u, hq_m):
    am = []
    for pair in ((0, 1), (2, 3)):
        a, b = _sc_agg_m(src4, dst4, iou, iom, hq_u[pair[0]], hq_u[pair[1]])
        am += [_assemble(a, M_ST, M_LA), _assemble(b, M_ST, M_LA)]
    au = []
    for pair in ((0, 1), (2, 3)):
        a, b = _sc_agg_u(dst4, src4, iom, iou, hq_m[pair[0]], hq_m[pair[1]])
        au += [_assemble(a, U_ST, U_LA), _assemble(b, U_ST, U_LA)]
    return tuple(au), tuple(am)


def kernel(x_user, x_movie, edge_src_user, edge_dst_movie, params):
    p = params
    r2 = lambda v: v.reshape(1, H)
    src4 = edge_src_user.astype(jnp.int32).reshape(16, NBLK, BLKROWS, CHUNK)
    dst4 = edge_dst_movie.astype(jnp.int32).reshape(16, NBLK, BLKROWS, CHUNK)
    iou = jnp.minimum(
        jnp.arange(16, dtype=jnp.int32)[:, None] * U_ST
        + jnp.arange(U_LA, dtype=jnp.int32)[None, :],
        N_USER - 1).reshape(16, U_ROWS, CHUNK)
    iom = jnp.minimum(
        jnp.arange(16, dtype=jnp.int32)[:, None] * M_ST
        + jnp.arange(M_ROWS * CHUNK, dtype=jnp.int32)[None, :],
        N_MOVIE - 1).reshape(16, M_ROWS, CHUNK)

    hq_u = _enc(x_user, p['W_ue'], r2(p['b_ue']), r2(p['g_ue']), r2(p['be_ue']))
    hq_m = _enc(x_movie, p['W_me'], r2(p['b_me']), r2(p['g_me']), r2(p['be_me']))
    cu3, cm3 = _sc_counts(src4, dst4, iou, iom)
    cu = _assemble(cu3, U_ST, U_LA)
    cm = _assemble(cm3, M_ST, M_LA)

    aq_u, aq_m = _agg_all(src4, dst4, iou, iom, hq_u, hq_m)
    h1q_u = _conv_dense(aq_u, cu, hq_u,
                        p['Wl1_u'], r2(p['bl1_u']), p['Wr1_u'],
                        r2(p['g1_u']), r2(p['b1_u']), relu=True, split=True)
    h1q_m = _conv_dense(aq_m, cm, hq_m,
                        p['Wl1_m'], r2(p['bl1_m']), p['Wr1_m'],
                        r2(p['g1_m']), r2(p['b1_m']), relu=True, split=True)

    bq_u, bq_m = _agg_all(src4, dst4, iou, iom, h1q_u, h1q_m)
    out_u = _conv_dense(bq_u, cu, h1q_u,
                        p['Wl2_u'], r2(p['bl2_u']), p['Wr2_u'],
                        r2(p['g2_u']), r2(p['b2_u']), relu=False, split=False)
    out_m = _conv_dense(bq_m, cm, h1q_m,
                        p['Wl2_m'], r2(p['bl2_m']), p['Wr2_m'],
                        r2(p['g2_m']), r2(p['b2_m']), relu=False, split=False)
    return out_u, out_m


# speedup vs baseline: 1.0976x; 1.0916x over previous
import functools

import jax
import jax.numpy as jnp
from jax import lax
from jax.experimental import pallas as pl
from jax.experimental.pallas import tpu as pltpu
from jax.experimental.pallas import tpu_sc as plsc

N_USER = 50000
N_MOVIE = 10000
E = 800000
H = 64
Q = 16

CHUNK = 80
ECH = 125
NBLK = 5
BLKROWS = 80

U_ST, U_LA = 3120, 3200
M_ST, M_LA = 640, 400
U_ROWS = U_LA // CHUNK
M_ROWS = M_LA * 0 + 8

_f32 = jnp.float32


def _fill_rows(ref, nrows, width, vec16):
    for r in range(nrows):
        for h in range(width // 16):
            ref[r, pl.ds(h * 16, 16)] = vec16


def _per_tile(s, st, la, fn):
    @pl.when(s < 15)
    def _():
        fn(s * st, st // CHUNK)

    @pl.when(s == 15)
    def _():
        fn(15 * st, la // CHUNK)


def _ind_zero(zbuf, sh, iot, nch):
    for k in range(nch):
        pltpu.sync_copy(zbuf, sh.at[iot.at[k]])


def _ind_stage(vbuf, hbm, sh, iot, base, nch):
    for k in range(nch):
        pltpu.sync_copy(hbm.at[pl.ds(base + k * CHUNK, CHUNK)], vbuf)
        pltpu.sync_copy(vbuf, sh.at[iot.at[k]])


def _ind_drain(vbuf, sh, out3, iot, s, nch, sem):
    for k in range(nch):
        pltpu.async_copy(sh.at[iot.at[k]], vbuf, sem).wait()
        pltpu.sync_copy(vbuf, out3.at[s, pl.ds(k * CHUNK, CHUNK)])


def _counts_body(src4, dst4, iou, iom, cu3, cm3,
                 cu_sh, cm_sh, idxb, iotu, iotm, ones, vbuf, sem):
    c = lax.axis_index("c")
    s = lax.axis_index("s")
    one16 = jnp.ones((16,), _f32)
    zero16 = jnp.zeros((16,), _f32)
    _fill_rows(ones, ECH, 16, one16)
    _fill_rows(vbuf, CHUNK, 16, zero16)
    pltpu.sync_copy(iou.at[s], iotu)
    pltpu.sync_copy(iom.at[s], iotm)

    @pl.when(c == 0)
    def _():
        _per_tile(s, U_ST, U_LA, lambda b, n: _ind_zero(vbuf, cu_sh, iotu, n))

    @pl.when(c == 1)
    def _():
        _per_tile(s, M_ST, M_LA, lambda b, n: _ind_zero(vbuf, cm_sh, iotm, n))
    plsc.subcore_barrier()

    def scatter(idx_hbm, cnt_sh):
        def blk_body(blk, carry):
            pltpu.sync_copy(idx_hbm.at[s, blk], idxb)
            for j in range(BLKROWS):
                pltpu.sync_copy(ones, cnt_sh.at[idxb.at[j]], add=True)
            return carry
        lax.fori_loop(0, NBLK, blk_body, 0)

    @pl.when(c == 0)
    def _():
        scatter(src4, cu_sh)

    @pl.when(c == 1)
    def _():
        scatter(dst4, cm_sh)
    plsc.subcore_barrier()

    @pl.when(c == 0)
    def _():
        _per_tile(s, U_ST, U_LA,
                  lambda b, n: _ind_drain(vbuf, cu_sh, cu3, iotu, s, n, sem))

    @pl.when(c == 1)
    def _():
        _per_tile(s, M_ST, M_LA,
                  lambda b, n: _ind_drain(vbuf, cm_sh, cm3, iotm, s, n, sem))


@functools.partial(
    pl.kernel,
    out_type=[jax.ShapeDtypeStruct((16, U_LA, 16), _f32),
              jax.ShapeDtypeStruct((16, M_ROWS * CHUNK, 16), _f32)],
    mesh=plsc.VectorSubcoreMesh(core_axis_name="c", subcore_axis_name="s"),
    scratch_types=[
        pltpu.VMEM_SHARED((N_USER, 16), _f32),
        pltpu.VMEM_SHARED((N_MOVIE, 16), _f32),
        pltpu.VMEM((BLKROWS, ECH), jnp.int32),
        pltpu.VMEM((U_ROWS, CHUNK), jnp.int32),
        pltpu.VMEM((M_ROWS, CHUNK), jnp.int32),
        pltpu.VMEM((ECH, 16), _f32),
        pltpu.VMEM((CHUNK, 16), _f32),
        pltpu.SemaphoreType.DMA,
    ],
)
def _sc_counts(src4, dst4, iou, iom, cu3, cm3,
               cu_sh, cm_sh, idxb, iotu, iotm, ones, vbuf, sem):
    _counts_body(src4, dst4, iou, iom, cu3, cm3,
                 cu_sh, cm_sh, idxb, iotu, iotm, ones, vbuf, sem)


def _dir_body(gidx_hbm, sidx_hbm, io_t, io_a, tq_a, tq_b, acc_a, acc_b,
              tbl_sh, acc_sh, gidx, sidx, iott, iota, buf, buf2, vbuf,
              sem0, sem1, sem2,
              t_st, t_la, a_st, a_la):
    c = lax.axis_index("c")
    s = lax.axis_index("s")
    zero16 = jnp.zeros((16,), _f32)
    _fill_rows(vbuf, CHUNK, Q, zero16)
    pltpu.sync_copy(io_t.at[s], iott)
    pltpu.sync_copy(io_a.at[s], iota)

    _per_tile(s, a_st, a_la, lambda b, n: _ind_zero(vbuf, acc_sh, iota, n))

    @pl.when(c == 0)
    def _():
        _per_tile(s, t_st, t_la,
                  lambda b, n: _ind_stage(vbuf, tq_a, tbl_sh, iott, b, n))

    @pl.when(c == 1)
    def _():
        _per_tile(s, t_st, t_la,
                  lambda b, n: _ind_stage(vbuf, tq_b, tbl_sh, iott, b, n))
    plsc.subcore_barrier()

    bufs = (buf, buf2)
    sems = (sem0, sem2)

    def blk_body(blk, carry):
        pltpu.sync_copy(gidx_hbm.at[s, blk], gidx)
        pltpu.sync_copy(sidx_hbm.at[s, blk], sidx)
        pltpu.async_copy(tbl_sh.at[gidx.at[0]], bufs[0], sems[0])
        for j in range(BLKROWS):
            b = j % 2
            pltpu.make_async_copy(tbl_sh.at[gidx.at[j]], bufs[b],
                                  sems[b]).wait()
            if j + 1 < BLKROWS:
                pltpu.async_copy(tbl_sh.at[gidx.at[j + 1]], bufs[1 - b],
                                 sems[1 - b])
            pltpu.sync_copy(bufs[b], acc_sh.at[sidx.at[j]], add=True)
        return carry
    lax.fori_loop(0, NBLK, blk_body, 0)
    plsc.subcore_barrier()

    @pl.when(c == 0)
    def _():
        _per_tile(s, a_st, a_la,
                  lambda b, n: _ind_drain(vbuf, acc_sh, acc_a, iota, s, n, sem1))

    @pl.when(c == 1)
    def _():
        _per_tile(s, a_st, a_la,
                  lambda b, n: _ind_drain(vbuf, acc_sh, acc_b, iota, s, n, sem1))


def _make_dir_kernel(tbl_rows, acc_rows, t_stripes, a_stripes, a_rows, t_rows):
    @functools.partial(
        pl.kernel,
        out_type=[jax.ShapeDtypeStruct((16, a_rows * CHUNK, Q), _f32),
                  jax.ShapeDtypeStruct((16, a_rows * CHUNK, Q), _f32)],
        mesh=plsc.VectorSubcoreMesh(core_axis_name="c", subcore_axis_name="s"),
        scratch_types=[
            pltpu.VMEM_SHARED((tbl_rows, Q), _f32),
            pltpu.VMEM_SHARED((acc_rows, Q), _f32),
            pltpu.VMEM((BLKROWS, ECH), jnp.int32),
            pltpu.VMEM((BLKROWS, ECH), jnp.int32),
            pltpu.VMEM((t_rows, CHUNK), jnp.int32),
            pltpu.VMEM((a_rows, CHUNK), jnp.int32),
            pltpu.VMEM((ECH, Q), _f32),
            pltpu.VMEM((ECH, Q), _f32),
            pltpu.VMEM((CHUNK, Q), _f32),
            pltpu.SemaphoreType.DMA,
            pltpu.SemaphoreType.DMA,
            pltpu.SemaphoreType.DMA,
        ],
    )
    def _k(gidx_hbm, sidx_hbm, io_t, io_a, tq_a, tq_b, acc_a, acc_b,
           tbl_sh, acc_sh, gidx, sidx, iott, iota, buf, buf2, vbuf,
           sem0, sem1, sem2):
        _dir_body(gidx_hbm, sidx_hbm, io_t, io_a, tq_a, tq_b, acc_a, acc_b,
                  tbl_sh, acc_sh, gidx, sidx, iott, iota, buf, buf2, vbuf,
                  sem0, sem1, sem2,
                  t_stripes[0], t_stripes[1], a_stripes[0], a_stripes[1])
    return _k


_sc_agg_m = _make_dir_kernel(N_USER, N_MOVIE, (U_ST, U_LA), (M_ST, M_LA),
                             M_ROWS, U_ROWS)
_sc_agg_u = _make_dir_kernel(N_MOVIE, N_USER, (M_ST, M_LA), (U_ST, U_LA),
                             U_ROWS, M_ROWS)


_BN = 1000


def _ln(o, g, b):
    m = jnp.mean(o, axis=-1, keepdims=True)
    v = jnp.mean((o - m) ** 2, axis=-1, keepdims=True)
    return (o - m) / jnp.sqrt(v + 1e-5) * g + b


def _q_split_store(on, outs):
    for k in range(4):
        outs[k][...] = on[:, k * Q:(k + 1) * Q]


def _enc(x, W, b, g, be):
    N, F = x.shape

    def body(x_ref, w_ref, b_ref, g_ref, be_ref, *outs):
        h = jnp.dot(x_ref[...], w_ref[...], preferred_element_type=_f32)
        h = jax.nn.relu(h + b_ref[...])
        hn = _ln(h, g_ref[...], be_ref[...])
        _q_split_store(hn, outs)

    return pl.pallas_call(
        body,
        grid=(N // _BN,),
        in_specs=[
            pl.BlockSpec((_BN, F), lambda i: (i, 0)),
            pl.BlockSpec((F, H), lambda i: (0, 0)),
            pl.BlockSpec((1, H), lambda i: (0, 0)),
            pl.BlockSpec((1, H), lambda i: (0, 0)),
            pl.BlockSpec((1, H), lambda i: (0, 0)),
        ],
        out_specs=[pl.BlockSpec((_BN, Q), lambda i: (i, 0))] * 4,
        out_shape=[jax.ShapeDtypeStruct((N, Q), _f32)] * 4,
    )(x, W, b, g, be)


def _conv_dense(aq, cnt, hq, Wl, bl, Wr, g, b, relu, split):
    N = aq[0].shape[0]

    def body(a0r, a1r, a2r, a3r, cr, h0r, h1r, h2r, h3r,
             wlr, blr, wrr, gr, br, *outs):
        agg = jnp.concatenate([a0r[...], a1r[...], a2r[...], a3r[...]], axis=1)
        h = jnp.concatenate([h0r[...], h1r[...], h2r[...], h3r[...]], axis=1)
        c = cr[...][:, 0:1]
        mean = jnp.where(c > 0, agg / jnp.maximum(c, 1.0), 0.0)
        o = (jnp.dot(mean, wlr[...], preferred_element_type=_f32) + blr[...]
             + jnp.dot(h, wrr[...], preferred_element_type=_f32))
        on = _ln(o, gr[...], br[...])
        if relu:
            on = jax.nn.relu(on)
        if split:
            _q_split_store(on, outs)
        else:
            outs[0][...] = on

    if split:
        out_specs = [pl.BlockSpec((_BN, Q), lambda i: (i, 0))] * 4
        out_shape = [jax.ShapeDtypeStruct((N, Q), _f32)] * 4
    else:
        out_specs = [pl.BlockSpec((_BN, H), lambda i: (i, 0))]
        out_shape = [jax.ShapeDtypeStruct((N, H), _f32)]

    res = pl.pallas_call(
        body,
        grid=(N // _BN,),
        in_specs=(
            [pl.BlockSpec((_BN, Q), lambda i: (i, 0))] * 4
            + [pl.BlockSpec((_BN, 16), lambda i: (i, 0))]
            + [pl.BlockSpec((_BN, Q), lambda i: (i, 0))] * 4
            + [pl.BlockSpec((H, H), lambda i: (0, 0)),
               pl.BlockSpec((1, H), lambda i: (0, 0)),
               pl.BlockSpec((H, H), lambda i: (0, 0)),
               pl.BlockSpec((1, H), lambda i: (0, 0)),
               pl.BlockSpec((1, H), lambda i: (0, 0))]
        ),
        out_specs=out_specs,
        out_shape=out_shape,
    )(*aq, cnt, *hq, Wl, bl, Wr, g, b)
    return res if split else res[0]


def _assemble(o3, st, la):
    parts = [o3[t, :st] for t in range(15)] + [o3[15, :la]]
    return jnp.concatenate(parts, axis=0)


def _agg_all(src4, dst4, iou, iom, hq_u, hq_m):
    am = []
    for pair in ((0, 1), (2, 3)):
        a, b = _sc_agg_m(src4, dst4, iou, iom, hq_u[pair[0]], hq_u[pair[1]])
        am += [_assemble(a, M_ST, M_LA), _assemble(b, M_ST, M_LA)]
    au = []
    for pair in ((0, 1), (2, 3)):
        a, b = _sc_agg_u(dst4, src4, iom, iou, hq_m[pair[0]], hq_m[pair[1]])
        au += [_assemble(a, U_ST, U_LA), _assemble(b, U_ST, U_LA)]
    return tuple(au), tuple(am)


def kernel(x_user, x_movie, edge_src_user, edge_dst_movie, params):
    p = params
    r2 = lambda v: v.reshape(1, H)
    src4 = edge_src_user.astype(jnp.int32).reshape(16, NBLK, BLKROWS, ECH)
    dst4 = edge_dst_movie.astype(jnp.int32).reshape(16, NBLK, BLKROWS, ECH)
    iou = jnp.minimum(
        jnp.arange(16, dtype=jnp.int32)[:, None] * U_ST
        + jnp.arange(U_LA, dtype=jnp.int32)[None, :],
        N_USER - 1).reshape(16, U_ROWS, CHUNK)
    iom = jnp.minimum(
        jnp.arange(16, dtype=jnp.int32)[:, None] * M_ST
        + jnp.arange(M_ROWS * CHUNK, dtype=jnp.int32)[None, :],
        N_MOVIE - 1).reshape(16, M_ROWS, CHUNK)

    hq_u = _enc(x_user, p['W_ue'], r2(p['b_ue']), r2(p['g_ue']), r2(p['be_ue']))
    hq_m = _enc(x_movie, p['W_me'], r2(p['b_me']), r2(p['g_me']), r2(p['be_me']))
    cu3, cm3 = _sc_counts(src4, dst4, iou, iom)
    cu = _assemble(cu3, U_ST, U_LA)
    cm = _assemble(cm3, M_ST, M_LA)

    aq_u, aq_m = _agg_all(src4, dst4, iou, iom, hq_u, hq_m)
    h1q_u = _conv_dense(aq_u, cu, hq_u,
                        p['Wl1_u'], r2(p['bl1_u']), p['Wr1_u'],
                        r2(p['g1_u']), r2(p['b1_u']), relu=True, split=True)
    h1q_m = _conv_dense(aq_m, cm, hq_m,
                        p['Wl1_m'], r2(p['bl1_m']), p['Wr1_m'],
                        r2(p['g1_m']), r2(p['b1_m']), relu=True, split=True)

    bq_u, bq_m = _agg_all(src4, dst4, iou, iom, h1q_u, h1q_m)
    out_u = _conv_dense(bq_u, cu, h1q_u,
                        p['Wl2_u'], r2(p['bl2_u']), p['Wr2_u'],
                        r2(p['g2_u']), r2(p['b2_u']), relu=False, split=False)
    out_m = _conv_dense(bq_m, cm, h1q_m,
                        p['Wl2_m'], r2(p['bl2_m']), p['Wr2_m'],
                        r2(p['g2_m']), r2(p['b2_m']), relu=False, split=False)
    return out_u, out_m
